# baseline (speedup 1.0000x reference)
import functools

import jax
import jax.numpy as jnp
from jax import lax
from jax.experimental import pallas as pl
from jax.experimental.pallas import tpu as pltpu
from jax.experimental.pallas import tpu_sc as plsc

E = 8
H = 2048
F = 1408
T = 8192
EL = 128

M = 256
MSH = 8
R = 2 * T + E * M
NB = R // M
NBP = 80

HW = H // 2
NC, NS, L = 2, 16, 16
NW = NC * NS
TPW = T // NW
GRP = TPW // L

BT = 512
BC = 256


def _router_body(x_ref, gw_ref, eid_ref, rnk_ref, w_ref, cnt_ref, carry_ref):
    step = pl.program_id(0)

    @pl.when(step == 0)
    def _():
        carry_ref[...] = jnp.zeros_like(carry_ref)

    x = x_ref[...]
    logits = lax.dot_general(x, gw_ref[...], (((1,), (0,)), ((), ())),
                             preferred_element_type=jnp.float32)
    lane = lax.broadcasted_iota(jnp.int32, (BT, EL), 1)
    logits = jnp.where(lane < E, logits, -1e30)
    mx = jnp.max(logits, axis=1, keepdims=True)
    ex = jnp.exp(logits - mx)
    p = ex / jnp.sum(ex, axis=1, keepdims=True)

    m0 = jnp.max(p, axis=1, keepdims=True)
    i0 = jnp.min(jnp.where(p >= m0, lane, EL), axis=1, keepdims=True)
    oh0 = lane == i0
    p1 = jnp.where(oh0, -1.0, p)
    m1 = jnp.max(p1, axis=1, keepdims=True)
    i1 = jnp.min(jnp.where(p1 >= m1, lane, EL), axis=1, keepdims=True)
    oh1 = lane == i1
    s = m0 + m1

    ohb = (oh0 | oh1).astype(jnp.float32)
    r = lax.broadcasted_iota(jnp.int32, (BT, BT), 0)
    c = lax.broadcasted_iota(jnp.int32, (BT, BT), 1)
    tri = (c < r).astype(jnp.float32)
    excl = lax.dot_general(tri, ohb, (((1,), (0,)), ((), ())),
                           preferred_element_type=jnp.float32)
    carry = carry_ref[...]
    tot = excl + carry
    rank0 = jnp.sum(jnp.where(oh0, tot, 0.0), axis=1, keepdims=True)
    rank1 = jnp.sum(jnp.where(oh1, tot, 0.0), axis=1, keepdims=True)
    new_carry = carry + jnp.sum(ohb, axis=0, keepdims=True)
    carry_ref[...] = new_carry

    eid_ref[...] = jnp.concatenate([i0, i1], axis=1)
    rnk_ref[...] = jnp.concatenate([rank0, rank1], axis=1).astype(jnp.int32)
    w_ref[...] = jnp.concatenate([m0 / s, m1 / s], axis=1)
    cnt_ref[...] = new_carry[:, :L].astype(jnp.int32)


def _router(x, gw_pad):
    return pl.pallas_call(
        _router_body,
        grid=(T // BT,),
        in_specs=[
            pl.BlockSpec((BT, H), lambda i: (i, 0)),
            pl.BlockSpec((H, EL), lambda i: (0, 0)),
        ],
        out_specs=[
            pl.BlockSpec((BT, 2), lambda i: (i, 0)),
            pl.BlockSpec((BT, 2), lambda i: (i, 0)),
            pl.BlockSpec((BT, 2), lambda i: (i, 0)),
            pl.BlockSpec((1, L), lambda i: (0, 0)),
        ],
        out_shape=[
            jax.ShapeDtypeStruct((T, 2), jnp.int32),
            jax.ShapeDtypeStruct((T, 2), jnp.int32),
            jax.ShapeDtypeStruct((T, 2), jnp.float32),
            jax.ShapeDtypeStruct((1, L), jnp.int32),
        ],
        scratch_shapes=[pltpu.VMEM((1, EL), jnp.float32)],
    )(x, gw_pad)


def _dispatch_body(counts_hbm, e_hbm, r_hbm, hidden_hbm,
                   xs_hbm, d0_hbm, d1_hbm, be_hbm,
                   cnt_v, rs_v, bs_v, e_v, r_v, d0_v, d1_v, tok_v, be_v,
                   sem0, sem1, sem2, sem3, sem4, sem5):
    wid = lax.axis_index("s") * NC + lax.axis_index("c")
    base_t = wid * TPW
    base_a = base_t * 2

    pltpu.sync_copy(counts_hbm, cnt_v)
    iota = lax.iota(jnp.int32, L)
    cnt = cnt_v[...]
    nblk = lax.shift_right_logical(cnt + (M - 1), MSH)
    cnt_v[...] = nblk
    bexcl = jnp.zeros((L,), jnp.int32)
    for e in range(E):
        sp = plsc.load_gather(cnt_v, [jnp.full((L,), e, jnp.int32)])
        bexcl = bexcl + jnp.where(iota > e, sp, 0)
    bs_v[...] = bexcl
    rs_v[...] = lax.shift_left(bexcl, MSH)

    pltpu.sync_copy(e_hbm.at[pl.ds(base_a, 2 * TPW)], e_v)
    pltpu.sync_copy(r_hbm.at[pl.ds(base_a, 2 * TPW)], r_v)
    for j in range(GRP):
        a0 = (16 * j + iota) * 2
        e0 = plsc.load_gather(e_v, [a0])
        e1 = plsc.load_gather(e_v, [a0 + 1])
        r0 = plsc.load_gather(r_v, [a0])
        r1 = plsc.load_gather(r_v, [a0 + 1])
        d0_v[j] = plsc.load_gather(rs_v, [e0]) + r0
        d1_v[j] = plsc.load_gather(rs_v, [e1]) + r1

    pltpu.sync_copy(d0_v, d0_hbm.at[pl.ds(wid * GRP, GRP), :])
    pltpu.sync_copy(d1_v, d1_hbm.at[pl.ds(wid * GRP, GRP), :])

    tok0, tok1 = tok_v.at[0], tok_v.at[1]

    def fire_load(j, buf, sem):
        pltpu.async_copy(hidden_hbm.at[pl.ds(base_t + L * j, L), :], buf, sem)

    def drain_load(buf, sem):
        pltpu.make_async_copy(hidden_hbm.at[pl.ds(0, L), :], buf, sem).wait()

    fire_load(0, tok0, sem0)
    fire_load(1, tok1, sem1)

    def body(q, carry):
        j0 = 2 * q
        drain_load(tok0, sem0)
        c0 = pltpu.async_copy(tok0, xs_hbm.at[d0_v.at[j0]], sem2)
        c1 = pltpu.async_copy(tok0, xs_hbm.at[d1_v.at[j0]], sem3)
        drain_load(tok1, sem1)
        c2 = pltpu.async_copy(tok1, xs_hbm.at[d0_v.at[j0 + 1]], sem4)
        c3 = pltpu.async_copy(tok1, xs_hbm.at[d1_v.at[j0 + 1]], sem5)
        c0.wait()
        c1.wait()

        @pl.when(j0 + 2 < GRP)
        def _():
            fire_load(j0 + 2, tok0, sem0)

        c2.wait()
        c3.wait()

        @pl.when(j0 + 3 < GRP)
        def _():
            fire_load(j0 + 3, tok1, sem1)

        return carry

    lax.fori_loop(0, GRP // 2, body, 0)

    @pl.when(wid == 0)
    def _():
        for j in range(NBP // L):
            bi = iota + L * j
            acc = jnp.zeros((L,), jnp.int32)
            for e in range(E):
                bse = plsc.load_gather(bs_v, [jnp.full((L,), e, jnp.int32)])
                acc = acc + jnp.where(bi >= bse, 1, 0)
            be_v[pl.ds(L * j, L)] = acc - 1
        pltpu.sync_copy(be_v, be_hbm)


def _dispatch(counts16, e_flat, r_flat, hidden):
    mesh = plsc.VectorSubcoreMesh(core_axis_name="c", subcore_axis_name="s")
    return pl.kernel(
        _dispatch_body,
        out_type=[
            jax.ShapeDtypeStruct((R, HW), jnp.int32),
            jax.ShapeDtypeStruct((T // L, L), jnp.int32),
            jax.ShapeDtypeStruct((T // L, L), jnp.int32),
            jax.ShapeDtypeStruct((NBP,), jnp.int32),
        ],
        mesh=mesh,
        scratch_types=[
            pltpu.VMEM((L,), jnp.int32),
            pltpu.VMEM((L,), jnp.int32),
            pltpu.VMEM((L,), jnp.int32),
            pltpu.VMEM((2 * TPW,), jnp.int32),
            pltpu.VMEM((2 * TPW,), jnp.int32),
            pltpu.VMEM((GRP, L), jnp.int32),
            pltpu.VMEM((GRP, L), jnp.int32),
            pltpu.VMEM((2, L, HW), jnp.int32),
            pltpu.VMEM((NBP,), jnp.int32),
            pltpu.SemaphoreType.DMA,
            pltpu.SemaphoreType.DMA,
            pltpu.SemaphoreType.DMA,
            pltpu.SemaphoreType.DMA,
            pltpu.SemaphoreType.DMA,
            pltpu.SemaphoreType.DMA,
        ],
        compiler_params=pltpu.CompilerParams(needs_layout_passes=False),
    )(counts16, e_flat, r_flat, hidden)


def _ffn1_body(be_ref, x_ref, w1g_ref, w1u_ref, act_ref):
    x = x_ref[...].astype(jnp.float32)
    g = lax.dot_general(x, w1g_ref[0, 0], (((1,), (1,)), ((), ())),
                        preferred_element_type=jnp.float32)
    u = lax.dot_general(x, w1u_ref[0, 0], (((1,), (1,)), ((), ())),
                        preferred_element_type=jnp.float32)
    act_ref[...] = (g * u / (1.0 + jnp.exp(-g))).astype(jnp.bfloat16)


def _ffn1(be, xs, w1):
    w1r = w1.reshape(E, 2, F, H)
    return pl.pallas_call(
        _ffn1_body,
        grid_spec=pltpu.PrefetchScalarGridSpec(
            num_scalar_prefetch=1,
            grid=(NB,),
            in_specs=[
                pl.BlockSpec((M, H), lambda b, be: (b, 0)),
                pl.BlockSpec((1, 1, F, H), lambda b, be: (be[b], 0, 0, 0)),
                pl.BlockSpec((1, 1, F, H), lambda b, be: (be[b], 1, 0, 0)),
            ],
            out_specs=pl.BlockSpec((M, F), lambda b, be: (b, 0)),
        ),
        out_shape=jax.ShapeDtypeStruct((R, F), jnp.bfloat16),
        compiler_params=pltpu.CompilerParams(
            dimension_semantics=("arbitrary",),
        ),
    )(be, xs, w1r, w1r)


def _ffn2_body(be_ref, a_ref, w2_ref, y_ref):
    y_ref[...] = lax.dot_general(a_ref[...].astype(jnp.float32), w2_ref[0],
                                 (((1,), (1,)), ((), ())),
                                 preferred_element_type=jnp.float32
                                 ).astype(jnp.bfloat16)


def _ffn2(be, act, w2):
    return pl.pallas_call(
        _ffn2_body,
        grid_spec=pltpu.PrefetchScalarGridSpec(
            num_scalar_prefetch=1,
            grid=(NB,),
            in_specs=[
                pl.BlockSpec((M, F), lambda b, be: (b, 0)),
                pl.BlockSpec((1, H, F), lambda b, be: (be[b], 0, 0)),
            ],
            out_specs=pl.BlockSpec((M, H), lambda b, be: (b, 0)),
        ),
        out_shape=jax.ShapeDtypeStruct((R, H), jnp.bfloat16),
        compiler_params=pltpu.CompilerParams(
            dimension_semantics=("arbitrary",),
        ),
    )(be, act, w2)


def _unsort_body(ys_hbm, d0_hbm, d1_hbm, yu_hbm, d_v, b0, b1, sg0, sg1):
    wid = lax.axis_index("s") * NC + lax.axis_index("c")
    base_t = wid * TPW
    ngrp = 2 * GRP

    pltpu.sync_copy(d0_hbm.at[pl.ds(wid * GRP, GRP), :], d_v.at[0])
    pltpu.sync_copy(d1_hbm.at[pl.ds(wid * GRP, GRP), :], d_v.at[1])

    def out_slice(g):
        k = lax.shift_right_logical(g, 4)
        j = lax.bitwise_and(g, GRP - 1)
        return yu_hbm.at[pl.ds(k * T + base_t + L * j, L), :]

    def idx(g):
        k = lax.shift_right_logical(g, 4)
        j = lax.bitwise_and(g, GRP - 1)
        return d_v.at[k, j]

    def fire(g, buf, sem):
        pltpu.async_copy(ys_hbm.at[idx(g)], buf, sem)

    def drain(buf, sem):
        pltpu.make_async_copy(ys_hbm.at[d_v.at[0, 0]], buf, sem).wait()

    fire(0, b0, sg0)
    fire(1, b1, sg1)

    def body(q, carry):
        g0 = 2 * q
        drain(b0, sg0)
        w0 = pltpu.async_copy(b0, out_slice(g0), sg0)
        drain(b1, sg1)
        w1 = pltpu.async_copy(b1, out_slice(g0 + 1), sg1)
        w0.wait()

        @pl.when(g0 + 2 < ngrp)
        def _():
            fire(g0 + 2, b0, sg0)

        w1.wait()

        @pl.when(g0 + 3 < ngrp)
        def _():
            fire(g0 + 3, b1, sg1)

        return carry

    lax.fori_loop(0, GRP, body, 0)


def _unsort(ys, d0, d1):
    mesh = plsc.VectorSubcoreMesh(core_axis_name="c", subcore_axis_name="s")
    return pl.kernel(
        _unsort_body,
        out_type=jax.ShapeDtypeStruct((2 * T, HW), jnp.int32),
        mesh=mesh,
        scratch_types=[
            pltpu.VMEM((2, GRP, L), jnp.int32),
            pltpu.VMEM((L, HW), jnp.int32),
            pltpu.VMEM((L, HW), jnp.int32),
            pltpu.SemaphoreType.DMA,
            pltpu.SemaphoreType.DMA,
        ],
        compiler_params=pltpu.CompilerParams(needs_layout_passes=False),
    )(ys, d0, d1)


def _combine_body(y0_ref, y1_ref, w_ref, o_ref):
    w = w_ref[...]
    o_ref[...] = (y0_ref[...].astype(jnp.float32) * w[:, 0:1]
                  + y1_ref[...].astype(jnp.float32) * w[:, 1:2])


def _combine(yu, w):
    return pl.pallas_call(
        _combine_body,
        grid=(T // BC,),
        in_specs=[
            pl.BlockSpec((BC, H), lambda i: (i, 0)),
            pl.BlockSpec((BC, H), lambda i: (T // BC + i, 0)),
            pl.BlockSpec((BC, 2), lambda i: (i, 0)),
        ],
        out_specs=pl.BlockSpec((BC, H), lambda i: (i, 0)),
        out_shape=jax.ShapeDtypeStruct((T, H), jnp.float32),
    )(yu, yu, w)


def kernel(hidden_states, gate_w, w1, w2):
    gw_pad = jnp.pad(gate_w, ((0, 0), (0, EL - E)))
    eids, ranks, wts, counts = _router(hidden_states, gw_pad)
    e_flat = eids.reshape(2 * T)
    r_flat = ranks.reshape(2 * T)
    counts16 = counts.reshape(L)
    hidden_b16 = hidden_states.astype(jnp.bfloat16)
    hidden32 = lax.bitcast_convert_type(
        hidden_b16.reshape(T, HW, 2), jnp.int32)
    xs32, d0, d1, be = _dispatch(counts16, e_flat, r_flat, hidden32)
    xs = lax.bitcast_convert_type(xs32, jnp.bfloat16).reshape(R, H)
    act = _ffn1(be, xs, w1)
    ys = _ffn2(be, act, w2)
    ys32 = lax.bitcast_convert_type(ys.reshape(R, HW, 2), jnp.int32)
    yu32 = _unsort(ys32, d0, d1)
    yu = lax.bitcast_convert_type(yu32, jnp.bfloat16).reshape(2 * T, H)
    return _combine(yu, wts)


# speedup vs baseline: 3.6515x; 3.6515x over previous
import functools

import jax
import jax.numpy as jnp
from jax import lax
from jax.experimental import pallas as pl
from jax.experimental.pallas import tpu as pltpu
from jax.experimental.pallas import tpu_sc as plsc

E = 8
H = 2048
F = 1408
T = 8192
EL = 128

M = 256
MSH = 8
R = 2 * T + E * M
NB = R // M
NBP = 80

HW = H // 2
NC, NS, L = 2, 16, 16
NW = NC * NS
TPW = T // NW
GRP = TPW // L

BT = 512
BC = 256


def _router_body(x_ref, gw_ref, eid_ref, rnk_ref, w_ref, cnt_ref, carry_ref):
    step = pl.program_id(0)

    @pl.when(step == 0)
    def _():
        carry_ref[...] = jnp.zeros_like(carry_ref)

    x = x_ref[...]
    logits = lax.dot_general(x, gw_ref[...], (((1,), (0,)), ((), ())),
                             preferred_element_type=jnp.float32)
    lane = lax.broadcasted_iota(jnp.int32, (BT, EL), 1)
    logits = jnp.where(lane < E, logits, -1e30)
    mx = jnp.max(logits, axis=1, keepdims=True)
    ex = jnp.exp(logits - mx)
    p = ex / jnp.sum(ex, axis=1, keepdims=True)

    m0 = jnp.max(p, axis=1, keepdims=True)
    i0 = jnp.min(jnp.where(p >= m0, lane, EL), axis=1, keepdims=True)
    oh0 = lane == i0
    p1 = jnp.where(oh0, -1.0, p)
    m1 = jnp.max(p1, axis=1, keepdims=True)
    i1 = jnp.min(jnp.where(p1 >= m1, lane, EL), axis=1, keepdims=True)
    oh1 = lane == i1
    s = m0 + m1

    ohb = (oh0 | oh1).astype(jnp.float32)
    r = lax.broadcasted_iota(jnp.int32, (BT, BT), 0)
    c = lax.broadcasted_iota(jnp.int32, (BT, BT), 1)
    tri = (c < r).astype(jnp.float32)
    excl = lax.dot_general(tri, ohb, (((1,), (0,)), ((), ())),
                           preferred_element_type=jnp.float32)
    carry = carry_ref[...]
    tot = excl + carry
    rank0 = jnp.sum(jnp.where(oh0, tot, 0.0), axis=1, keepdims=True)
    rank1 = jnp.sum(jnp.where(oh1, tot, 0.0), axis=1, keepdims=True)
    new_carry = carry + jnp.sum(ohb, axis=0, keepdims=True)
    carry_ref[...] = new_carry

    eid_ref[...] = jnp.concatenate([i0, i1], axis=1)
    rnk_ref[...] = jnp.concatenate([rank0, rank1], axis=1).astype(jnp.int32)
    w_ref[...] = jnp.concatenate([m0 / s, m1 / s], axis=1)
    cnt_ref[...] = new_carry[:, :L].astype(jnp.int32)


def _router(x, gw_pad):
    return pl.pallas_call(
        _router_body,
        grid=(T // BT,),
        in_specs=[
            pl.BlockSpec((BT, H), lambda i: (i, 0)),
            pl.BlockSpec((H, EL), lambda i: (0, 0)),
        ],
        out_specs=[
            pl.BlockSpec((BT, 2), lambda i: (i, 0)),
            pl.BlockSpec((BT, 2), lambda i: (i, 0)),
            pl.BlockSpec((BT, 2), lambda i: (i, 0)),
            pl.BlockSpec((1, L), lambda i: (0, 0)),
        ],
        out_shape=[
            jax.ShapeDtypeStruct((T, 2), jnp.int32),
            jax.ShapeDtypeStruct((T, 2), jnp.int32),
            jax.ShapeDtypeStruct((T, 2), jnp.float32),
            jax.ShapeDtypeStruct((1, L), jnp.int32),
        ],
        scratch_shapes=[pltpu.VMEM((1, EL), jnp.float32)],
    )(x, gw_pad)


def _dispatch_body(counts_hbm, e_hbm, r_hbm, hidden_hbm,
                   xs_hbm, d0_hbm, d1_hbm, be_hbm,
                   cnt_v, rs_v, bs_v, e_v, r_v, d0_v, d1_v, tok_v, be_v,
                   sem0, sem1, sem2, sem3, sem4, sem5):
    wid = lax.axis_index("s") * NC + lax.axis_index("c")
    base_t = wid * TPW
    base_a = base_t * 2

    pltpu.sync_copy(counts_hbm, cnt_v)
    iota = lax.iota(jnp.int32, L)
    cnt = cnt_v[...]
    nblk = lax.shift_right_logical(cnt + (M - 1), MSH)
    cnt_v[...] = nblk
    bexcl = jnp.zeros((L,), jnp.int32)
    for e in range(E):
        sp = plsc.load_gather(cnt_v, [jnp.full((L,), e, jnp.int32)])
        bexcl = bexcl + jnp.where(iota > e, sp, 0)
    bs_v[...] = bexcl
    rs_v[...] = lax.shift_left(bexcl, MSH)

    pltpu.sync_copy(e_hbm.at[pl.ds(base_a, 2 * TPW)], e_v)
    pltpu.sync_copy(r_hbm.at[pl.ds(base_a, 2 * TPW)], r_v)
    for j in range(GRP):
        a0 = (16 * j + iota) * 2
        e0 = plsc.load_gather(e_v, [a0])
        e1 = plsc.load_gather(e_v, [a0 + 1])
        r0 = plsc.load_gather(r_v, [a0])
        r1 = plsc.load_gather(r_v, [a0 + 1])
        d0_v[j] = plsc.load_gather(rs_v, [e0]) + r0
        d1_v[j] = plsc.load_gather(rs_v, [e1]) + r1

    pltpu.sync_copy(d0_v, d0_hbm.at[pl.ds(wid * GRP, GRP), :])
    pltpu.sync_copy(d1_v, d1_hbm.at[pl.ds(wid * GRP, GRP), :])

    tok0, tok1 = tok_v.at[0], tok_v.at[1]

    def fire_load(j, buf, sem):
        pltpu.async_copy(hidden_hbm.at[pl.ds(base_t + L * j, L), :], buf, sem)

    def drain_load(buf, sem):
        pltpu.make_async_copy(hidden_hbm.at[pl.ds(0, L), :], buf, sem).wait()

    fire_load(0, tok0, sem0)
    fire_load(1, tok1, sem1)

    def body(q, carry):
        j0 = 2 * q
        drain_load(tok0, sem0)
        c0 = pltpu.async_copy(tok0, xs_hbm.at[d0_v.at[j0]], sem2)
        c1 = pltpu.async_copy(tok0, xs_hbm.at[d1_v.at[j0]], sem3)
        drain_load(tok1, sem1)
        c2 = pltpu.async_copy(tok1, xs_hbm.at[d0_v.at[j0 + 1]], sem4)
        c3 = pltpu.async_copy(tok1, xs_hbm.at[d1_v.at[j0 + 1]], sem5)
        c0.wait()
        c1.wait()

        @pl.when(j0 + 2 < GRP)
        def _():
            fire_load(j0 + 2, tok0, sem0)

        c2.wait()
        c3.wait()

        @pl.when(j0 + 3 < GRP)
        def _():
            fire_load(j0 + 3, tok1, sem1)

        return carry

    lax.fori_loop(0, GRP // 2, body, 0)

    @pl.when(wid == 0)
    def _():
        for j in range(NBP // L):
            bi = iota + L * j
            acc = jnp.zeros((L,), jnp.int32)
            for e in range(E):
                bse = plsc.load_gather(bs_v, [jnp.full((L,), e, jnp.int32)])
                acc = acc + jnp.where(bi >= bse, 1, 0)
            be_v[pl.ds(L * j, L)] = acc - 1
        pltpu.sync_copy(be_v, be_hbm)


def _dispatch(counts16, e_flat, r_flat, hidden):
    mesh = plsc.VectorSubcoreMesh(core_axis_name="c", subcore_axis_name="s")
    return pl.kernel(
        _dispatch_body,
        out_type=[
            jax.ShapeDtypeStruct((R, H), jnp.float32),
            jax.ShapeDtypeStruct((T // L, L), jnp.int32),
            jax.ShapeDtypeStruct((T // L, L), jnp.int32),
            jax.ShapeDtypeStruct((NBP,), jnp.int32),
        ],
        mesh=mesh,
        scratch_types=[
            pltpu.VMEM((L,), jnp.int32),
            pltpu.VMEM((L,), jnp.int32),
            pltpu.VMEM((L,), jnp.int32),
            pltpu.VMEM((2 * TPW,), jnp.int32),
            pltpu.VMEM((2 * TPW,), jnp.int32),
            pltpu.VMEM((GRP, L), jnp.int32),
            pltpu.VMEM((GRP, L), jnp.int32),
            pltpu.VMEM((2, L, H), jnp.float32),
            pltpu.VMEM((NBP,), jnp.int32),
            pltpu.SemaphoreType.DMA,
            pltpu.SemaphoreType.DMA,
            pltpu.SemaphoreType.DMA,
            pltpu.SemaphoreType.DMA,
            pltpu.SemaphoreType.DMA,
            pltpu.SemaphoreType.DMA,
        ],
        compiler_params=pltpu.CompilerParams(needs_layout_passes=False),
    )(counts16, e_flat, r_flat, hidden)


def _ffn1_body(be_ref, x_ref, w1g_ref, w1u_ref, act_ref):
    x = x_ref[...]
    g = lax.dot_general(x, w1g_ref[0, 0], (((1,), (1,)), ((), ())),
                        preferred_element_type=jnp.float32)
    u = lax.dot_general(x, w1u_ref[0, 0], (((1,), (1,)), ((), ())),
                        preferred_element_type=jnp.float32)
    act_ref[...] = g * u / (1.0 + jnp.exp(-g))


def _ffn1(be, xs, w1):
    w1r = w1.reshape(E, 2, F, H)
    return pl.pallas_call(
        _ffn1_body,
        grid_spec=pltpu.PrefetchScalarGridSpec(
            num_scalar_prefetch=1,
            grid=(NB,),
            in_specs=[
                pl.BlockSpec((M, H), lambda b, be: (b, 0)),
                pl.BlockSpec((1, 1, F, H), lambda b, be: (be[b], 0, 0, 0)),
                pl.BlockSpec((1, 1, F, H), lambda b, be: (be[b], 1, 0, 0)),
            ],
            out_specs=pl.BlockSpec((M, F), lambda b, be: (b, 0)),
        ),
        out_shape=jax.ShapeDtypeStruct((R, F), jnp.float32),
        compiler_params=pltpu.CompilerParams(
            dimension_semantics=("arbitrary",),
        ),
    )(be, xs, w1r, w1r)


def _ffn2_body(be_ref, a_ref, w2_ref, y_ref):
    y_ref[...] = lax.dot_general(a_ref[...], w2_ref[0],
                                 (((1,), (1,)), ((), ())),
                                 preferred_element_type=jnp.float32)


def _ffn2(be, act, w2):
    return pl.pallas_call(
        _ffn2_body,
        grid_spec=pltpu.PrefetchScalarGridSpec(
            num_scalar_prefetch=1,
            grid=(NB,),
            in_specs=[
                pl.BlockSpec((M, F), lambda b, be: (b, 0)),
                pl.BlockSpec((1, H, F), lambda b, be: (be[b], 0, 0)),
            ],
            out_specs=pl.BlockSpec((M, H), lambda b, be: (b, 0)),
        ),
        out_shape=jax.ShapeDtypeStruct((R, H), jnp.float32),
        compiler_params=pltpu.CompilerParams(
            dimension_semantics=("arbitrary",),
        ),
    )(be, act, w2)


def _unsort_body(ys_hbm, d0_hbm, d1_hbm, yu_hbm, d_v, b0, b1, sg0, sg1):
    wid = lax.axis_index("s") * NC + lax.axis_index("c")
    base_t = wid * TPW
    ngrp = 2 * GRP

    pltpu.sync_copy(d0_hbm.at[pl.ds(wid * GRP, GRP), :], d_v.at[0])
    pltpu.sync_copy(d1_hbm.at[pl.ds(wid * GRP, GRP), :], d_v.at[1])

    def out_slice(g):
        k = lax.shift_right_logical(g, 4)
        j = lax.bitwise_and(g, GRP - 1)
        return yu_hbm.at[pl.ds(k * T + base_t + L * j, L), :]

    def idx(g):
        k = lax.shift_right_logical(g, 4)
        j = lax.bitwise_and(g, GRP - 1)
        return d_v.at[k, j]

    def fire(g, buf, sem):
        pltpu.async_copy(ys_hbm.at[idx(g)], buf, sem)

    def drain(buf, sem):
        pltpu.make_async_copy(ys_hbm.at[d_v.at[0, 0]], buf, sem).wait()

    fire(0, b0, sg0)
    fire(1, b1, sg1)

    def body(q, carry):
        g0 = 2 * q
        drain(b0, sg0)
        w0 = pltpu.async_copy(b0, out_slice(g0), sg0)
        drain(b1, sg1)
        w1 = pltpu.async_copy(b1, out_slice(g0 + 1), sg1)
        w0.wait()

        @pl.when(g0 + 2 < ngrp)
        def _():
            fire(g0 + 2, b0, sg0)

        w1.wait()

        @pl.when(g0 + 3 < ngrp)
        def _():
            fire(g0 + 3, b1, sg1)

        return carry

    lax.fori_loop(0, GRP, body, 0)


def _unsort(ys, d0, d1):
    mesh = plsc.VectorSubcoreMesh(core_axis_name="c", subcore_axis_name="s")
    return pl.kernel(
        _unsort_body,
        out_type=jax.ShapeDtypeStruct((2 * T, H), jnp.float32),
        mesh=mesh,
        scratch_types=[
            pltpu.VMEM((2, GRP, L), jnp.int32),
            pltpu.VMEM((L, H), jnp.float32),
            pltpu.VMEM((L, H), jnp.float32),
            pltpu.SemaphoreType.DMA,
            pltpu.SemaphoreType.DMA,
        ],
        compiler_params=pltpu.CompilerParams(needs_layout_passes=False),
    )(ys, d0, d1)


def _combine_body(y0_ref, y1_ref, w_ref, o_ref):
    w = w_ref[...]
    o_ref[...] = y0_ref[...] * w[:, 0:1] + y1_ref[...] * w[:, 1:2]


def _combine(yu, w):
    return pl.pallas_call(
        _combine_body,
        grid=(T // BC,),
        in_specs=[
            pl.BlockSpec((BC, H), lambda i: (i, 0)),
            pl.BlockSpec((BC, H), lambda i: (T // BC + i, 0)),
            pl.BlockSpec((BC, 2), lambda i: (i, 0)),
        ],
        out_specs=pl.BlockSpec((BC, H), lambda i: (i, 0)),
        out_shape=jax.ShapeDtypeStruct((T, H), jnp.float32),
    )(yu, yu, w)


def kernel(hidden_states, gate_w, w1, w2):
    gw_pad = jnp.pad(gate_w, ((0, 0), (0, EL - E)))
    eids, ranks, wts, counts = _router(hidden_states, gw_pad)
    e_flat = eids.reshape(2 * T)
    r_flat = ranks.reshape(2 * T)
    counts16 = counts.reshape(L)
    xs, d0, d1, be = _dispatch(counts16, e_flat, r_flat, hidden_states)
    act = _ffn1(be, xs, w1)
    ys = _ffn2(be, act, w2)
    yu = _unsort(ys, d0, d1)
    return _combine(yu, wts)


# speedup vs baseline: 3.6540x; 1.0007x over previous
import functools

import jax
import jax.numpy as jnp
from jax import lax
from jax.experimental import pallas as pl
from jax.experimental.pallas import tpu as pltpu
from jax.experimental.pallas import tpu_sc as plsc

E = 8
H = 2048
F = 1408
T = 8192
EL = 128

M = 256
MSH = 8
R = 2 * T + E * M
NB = R // M
NBP = 80

HW = H // 2
NC, NS, L = 2, 16, 16
NW = NC * NS
TPW = T // NW
GRP = TPW // L

BT = 1024
BC = 512


def _router_body(x_ref, gw_ref, eid_ref, rnk_ref, w_ref, cnt_ref, carry_ref):
    step = pl.program_id(0)

    @pl.when(step == 0)
    def _():
        carry_ref[...] = jnp.zeros_like(carry_ref)

    x = x_ref[...]
    logits = lax.dot_general(x, gw_ref[...], (((1,), (0,)), ((), ())),
                             preferred_element_type=jnp.float32)
    lane = lax.broadcasted_iota(jnp.int32, (BT, EL), 1)
    logits = jnp.where(lane < E, logits, -1e30)
    mx = jnp.max(logits, axis=1, keepdims=True)
    ex = jnp.exp(logits - mx)
    p = ex / jnp.sum(ex, axis=1, keepdims=True)

    m0 = jnp.max(p, axis=1, keepdims=True)
    i0 = jnp.min(jnp.where(p >= m0, lane, EL), axis=1, keepdims=True)
    oh0 = lane == i0
    p1 = jnp.where(oh0, -1.0, p)
    m1 = jnp.max(p1, axis=1, keepdims=True)
    i1 = jnp.min(jnp.where(p1 >= m1, lane, EL), axis=1, keepdims=True)
    oh1 = lane == i1
    s = m0 + m1

    ohb = (oh0 | oh1).astype(jnp.float32)
    r = lax.broadcasted_iota(jnp.int32, (BT, BT), 0)
    c = lax.broadcasted_iota(jnp.int32, (BT, BT), 1)
    tri = (c < r).astype(jnp.float32)
    excl = lax.dot_general(tri, ohb, (((1,), (0,)), ((), ())),
                           preferred_element_type=jnp.float32)
    carry = carry_ref[...]
    tot = excl + carry
    rank0 = jnp.sum(jnp.where(oh0, tot, 0.0), axis=1, keepdims=True)
    rank1 = jnp.sum(jnp.where(oh1, tot, 0.0), axis=1, keepdims=True)
    new_carry = carry + jnp.sum(ohb, axis=0, keepdims=True)
    carry_ref[...] = new_carry

    eid_ref[...] = jnp.concatenate([i0, i1], axis=1)
    rnk_ref[...] = jnp.concatenate([rank0, rank1], axis=1).astype(jnp.int32)
    w_ref[...] = jnp.concatenate([m0 / s, m1 / s], axis=1)
    cnt_ref[...] = new_carry[:, :L].astype(jnp.int32)


def _router(x, gw_pad):
    return pl.pallas_call(
        _router_body,
        grid=(T // BT,),
        in_specs=[
            pl.BlockSpec((BT, H), lambda i: (i, 0)),
            pl.BlockSpec((H, EL), lambda i: (0, 0)),
        ],
        out_specs=[
            pl.BlockSpec((BT, 2), lambda i: (i, 0)),
            pl.BlockSpec((BT, 2), lambda i: (i, 0)),
            pl.BlockSpec((BT, 2), lambda i: (i, 0)),
            pl.BlockSpec((1, L), lambda i: (0, 0)),
        ],
        out_shape=[
            jax.ShapeDtypeStruct((T, 2), jnp.int32),
            jax.ShapeDtypeStruct((T, 2), jnp.int32),
            jax.ShapeDtypeStruct((T, 2), jnp.float32),
            jax.ShapeDtypeStruct((1, L), jnp.int32),
        ],
        scratch_shapes=[pltpu.VMEM((1, EL), jnp.float32)],
    )(x, gw_pad)


def _dispatch_body(counts_hbm, e_hbm, r_hbm, hidden_hbm,
                   xs_hbm, d0_hbm, d1_hbm, be_hbm,
                   cnt_v, rs_v, bs_v, e_v, r_v, d0_v, d1_v, tok_v, be_v,
                   sem0, sem1, sem2, sem3, sem4, sem5):
    wid = lax.axis_index("s") * NC + lax.axis_index("c")
    base_t = wid * TPW
    base_a = base_t * 2

    pltpu.sync_copy(counts_hbm, cnt_v)
    iota = lax.iota(jnp.int32, L)
    cnt = cnt_v[...]
    nblk = lax.shift_right_logical(cnt + (M - 1), MSH)
    cnt_v[...] = nblk
    bexcl = jnp.zeros((L,), jnp.int32)
    for e in range(E):
        sp = plsc.load_gather(cnt_v, [jnp.full((L,), e, jnp.int32)])
        bexcl = bexcl + jnp.where(iota > e, sp, 0)
    bs_v[...] = bexcl
    rs_v[...] = lax.shift_left(bexcl, MSH)

    pltpu.sync_copy(e_hbm.at[pl.ds(base_a, 2 * TPW)], e_v)
    pltpu.sync_copy(r_hbm.at[pl.ds(base_a, 2 * TPW)], r_v)
    for j in range(GRP):
        a0 = (16 * j + iota) * 2
        e0 = plsc.load_gather(e_v, [a0])
        e1 = plsc.load_gather(e_v, [a0 + 1])
        r0 = plsc.load_gather(r_v, [a0])
        r1 = plsc.load_gather(r_v, [a0 + 1])
        d0_v[j] = plsc.load_gather(rs_v, [e0]) + r0
        d1_v[j] = plsc.load_gather(rs_v, [e1]) + r1

    pltpu.sync_copy(d0_v, d0_hbm.at[pl.ds(wid * GRP, GRP), :])
    pltpu.sync_copy(d1_v, d1_hbm.at[pl.ds(wid * GRP, GRP), :])

    tok0, tok1 = tok_v.at[0], tok_v.at[1]

    def fire_load(j, buf, sem):
        pltpu.async_copy(hidden_hbm.at[pl.ds(base_t + L * j, L), :], buf, sem)

    def drain_load(buf, sem):
        pltpu.make_async_copy(hidden_hbm.at[pl.ds(0, L), :], buf, sem).wait()

    fire_load(0, tok0, sem0)
    fire_load(1, tok1, sem1)

    def body(q, carry):
        j0 = 2 * q
        drain_load(tok0, sem0)
        c0 = pltpu.async_copy(tok0, xs_hbm.at[d0_v.at[j0]], sem2)
        c1 = pltpu.async_copy(tok0, xs_hbm.at[d1_v.at[j0]], sem3)
        drain_load(tok1, sem1)
        c2 = pltpu.async_copy(tok1, xs_hbm.at[d0_v.at[j0 + 1]], sem4)
        c3 = pltpu.async_copy(tok1, xs_hbm.at[d1_v.at[j0 + 1]], sem5)
        c0.wait()
        c1.wait()

        @pl.when(j0 + 2 < GRP)
        def _():
            fire_load(j0 + 2, tok0, sem0)

        c2.wait()
        c3.wait()

        @pl.when(j0 + 3 < GRP)
        def _():
            fire_load(j0 + 3, tok1, sem1)

        return carry

    lax.fori_loop(0, GRP // 2, body, 0)

    @pl.when(wid == 0)
    def _():
        for j in range(NBP // L):
            bi = iota + L * j
            acc = jnp.zeros((L,), jnp.int32)
            for e in range(E):
                bse = plsc.load_gather(bs_v, [jnp.full((L,), e, jnp.int32)])
                acc = acc + jnp.where(bi >= bse, 1, 0)
            be_v[pl.ds(L * j, L)] = acc - 1
        pltpu.sync_copy(be_v, be_hbm)


def _dispatch(counts16, e_flat, r_flat, hidden):
    mesh = plsc.VectorSubcoreMesh(core_axis_name="c", subcore_axis_name="s")
    return pl.kernel(
        _dispatch_body,
        out_type=[
            jax.ShapeDtypeStruct((R, H), jnp.float32),
            jax.ShapeDtypeStruct((T // L, L), jnp.int32),
            jax.ShapeDtypeStruct((T // L, L), jnp.int32),
            jax.ShapeDtypeStruct((NBP,), jnp.int32),
        ],
        mesh=mesh,
        scratch_types=[
            pltpu.VMEM((L,), jnp.int32),
            pltpu.VMEM((L,), jnp.int32),
            pltpu.VMEM((L,), jnp.int32),
            pltpu.VMEM((2 * TPW,), jnp.int32),
            pltpu.VMEM((2 * TPW,), jnp.int32),
            pltpu.VMEM((GRP, L), jnp.int32),
            pltpu.VMEM((GRP, L), jnp.int32),
            pltpu.VMEM((2, L, H), jnp.float32),
            pltpu.VMEM((NBP,), jnp.int32),
            pltpu.SemaphoreType.DMA,
            pltpu.SemaphoreType.DMA,
            pltpu.SemaphoreType.DMA,
            pltpu.SemaphoreType.DMA,
            pltpu.SemaphoreType.DMA,
            pltpu.SemaphoreType.DMA,
        ],
        compiler_params=pltpu.CompilerParams(needs_layout_passes=False),
    )(counts16, e_flat, r_flat, hidden)


def _ffn1_body(be_ref, x_ref, w1g_ref, w1u_ref, act_ref):
    x = x_ref[...]
    g = lax.dot_general(x, w1g_ref[0, 0], (((1,), (1,)), ((), ())),
                        preferred_element_type=jnp.float32)
    u = lax.dot_general(x, w1u_ref[0, 0], (((1,), (1,)), ((), ())),
                        preferred_element_type=jnp.float32)
    act_ref[...] = g * u / (1.0 + jnp.exp(-g))


def _ffn1(be, xs, w1):
    w1r = w1.reshape(E, 2, F, H)
    return pl.pallas_call(
        _ffn1_body,
        grid_spec=pltpu.PrefetchScalarGridSpec(
            num_scalar_prefetch=1,
            grid=(NB,),
            in_specs=[
                pl.BlockSpec((M, H), lambda b, be: (b, 0)),
                pl.BlockSpec((1, 1, F, H), lambda b, be: (be[b], 0, 0, 0)),
                pl.BlockSpec((1, 1, F, H), lambda b, be: (be[b], 1, 0, 0)),
            ],
            out_specs=pl.BlockSpec((M, F), lambda b, be: (b, 0)),
        ),
        out_shape=jax.ShapeDtypeStruct((R, F), jnp.float32),
        compiler_params=pltpu.CompilerParams(
            dimension_semantics=("arbitrary",),
        ),
    )(be, xs, w1r, w1r)


def _ffn2_body(be_ref, a_ref, w2_ref, y_ref):
    y_ref[...] = lax.dot_general(a_ref[...], w2_ref[0],
                                 (((1,), (1,)), ((), ())),
                                 preferred_element_type=jnp.float32)


def _ffn2(be, act, w2):
    return pl.pallas_call(
        _ffn2_body,
        grid_spec=pltpu.PrefetchScalarGridSpec(
            num_scalar_prefetch=1,
            grid=(NB,),
            in_specs=[
                pl.BlockSpec((M, F), lambda b, be: (b, 0)),
                pl.BlockSpec((1, H, F), lambda b, be: (be[b], 0, 0)),
            ],
            out_specs=pl.BlockSpec((M, H), lambda b, be: (b, 0)),
        ),
        out_shape=jax.ShapeDtypeStruct((R, H), jnp.float32),
        compiler_params=pltpu.CompilerParams(
            dimension_semantics=("arbitrary",),
        ),
    )(be, act, w2)


def _unsort_body(ys_hbm, d0_hbm, d1_hbm, yu_hbm, d_v, b0, b1, sg0, sg1):
    wid = lax.axis_index("s") * NC + lax.axis_index("c")
    base_t = wid * TPW
    ngrp = 2 * GRP

    pltpu.sync_copy(d0_hbm.at[pl.ds(wid * GRP, GRP), :], d_v.at[0])
    pltpu.sync_copy(d1_hbm.at[pl.ds(wid * GRP, GRP), :], d_v.at[1])

    def out_slice(g):
        k = lax.shift_right_logical(g, 4)
        j = lax.bitwise_and(g, GRP - 1)
        return yu_hbm.at[pl.ds(k * T + base_t + L * j, L), :]

    def idx(g):
        k = lax.shift_right_logical(g, 4)
        j = lax.bitwise_and(g, GRP - 1)
        return d_v.at[k, j]

    def fire(g, buf, sem):
        pltpu.async_copy(ys_hbm.at[idx(g)], buf, sem)

    def drain(buf, sem):
        pltpu.make_async_copy(ys_hbm.at[d_v.at[0, 0]], buf, sem).wait()

    fire(0, b0, sg0)
    fire(1, b1, sg1)

    def body(q, carry):
        g0 = 2 * q
        drain(b0, sg0)
        w0 = pltpu.async_copy(b0, out_slice(g0), sg0)
        drain(b1, sg1)
        w1 = pltpu.async_copy(b1, out_slice(g0 + 1), sg1)
        w0.wait()

        @pl.when(g0 + 2 < ngrp)
        def _():
            fire(g0 + 2, b0, sg0)

        w1.wait()

        @pl.when(g0 + 3 < ngrp)
        def _():
            fire(g0 + 3, b1, sg1)

        return carry

    lax.fori_loop(0, GRP, body, 0)


def _unsort(ys, d0, d1):
    mesh = plsc.VectorSubcoreMesh(core_axis_name="c", subcore_axis_name="s")
    return pl.kernel(
        _unsort_body,
        out_type=jax.ShapeDtypeStruct((2 * T, H), jnp.float32),
        mesh=mesh,
        scratch_types=[
            pltpu.VMEM((2, GRP, L), jnp.int32),
            pltpu.VMEM((L, H), jnp.float32),
            pltpu.VMEM((L, H), jnp.float32),
            pltpu.SemaphoreType.DMA,
            pltpu.SemaphoreType.DMA,
        ],
        compiler_params=pltpu.CompilerParams(needs_layout_passes=False),
    )(ys, d0, d1)


def _combine_body(y0_ref, y1_ref, w_ref, o_ref):
    w = w_ref[...]
    o_ref[...] = y0_ref[...] * w[:, 0:1] + y1_ref[...] * w[:, 1:2]


def _combine(yu, w):
    return pl.pallas_call(
        _combine_body,
        grid=(T // BC,),
        in_specs=[
            pl.BlockSpec((BC, H), lambda i: (i, 0)),
            pl.BlockSpec((BC, H), lambda i: (T // BC + i, 0)),
            pl.BlockSpec((BC, 2), lambda i: (i, 0)),
        ],
        out_specs=pl.BlockSpec((BC, H), lambda i: (i, 0)),
        out_shape=jax.ShapeDtypeStruct((T, H), jnp.float32),
    )(yu, yu, w)


def kernel(hidden_states, gate_w, w1, w2):
    gw_pad = jnp.pad(gate_w, ((0, 0), (0, EL - E)))
    eids, ranks, wts, counts = _router(hidden_states, gw_pad)
    e_flat = eids.reshape(2 * T)
    r_flat = ranks.reshape(2 * T)
    counts16 = counts.reshape(L)
    xs, d0, d1, be = _dispatch(counts16, e_flat, r_flat, hidden_states)
    act = _ffn1(be, xs, w1)
    ys = _ffn2(be, act, w2)
    yu = _unsort(ys, d0, d1)
    return _combine(yu, wts)
